# baseline (speedup 1.0000x reference)
import functools

import jax
import jax.numpy as jnp
from jax import lax
from jax.experimental import pallas as pl
from jax.experimental.pallas import tpu as pltpu
from jax.experimental.pallas import tpu_sc as plsc

PRE_SEQ_LEN = 64
EMBED_DIM = 49152
BATCH = 32
NUM_ROWS = BATCH * PRE_SEQ_LEN
TBL_ROWS = PRE_SEQ_LEN

NUM_CORES = 2
NUM_SUBCORES = 16
LANES = 16

NUM_PHASES = 8
PHASE_W = EMBED_DIM // (NUM_CORES * NUM_PHASES)
PW_A = 2048
PW_B = PHASE_W - PW_A
STAGE_ROWS = TBL_ROWS // NUM_SUBCORES

ROWS_PER_WORKER = NUM_ROWS // NUM_SUBCORES
NUM_GROUPS = ROWS_PER_WORKER // LANES


def _sc_body(pref_hbm, tbl_hbm, out_hbm, idx_v, bbufs, spmem, asem, bsem, ssem, stsem):
    core = lax.axis_index("c")
    sub = lax.axis_index("s")
    base = sub * ROWS_PER_WORKER
    pltpu.sync_copy(pref_hbm.at[pl.ds(base, ROWS_PER_WORKER)], idx_v)
    srow = sub * STAGE_ROWS

    def stage(phase, pslot):
        pbase = (core * NUM_PHASES + phase) * PHASE_W
        return pltpu.make_async_copy(
            tbl_hbm.at[pl.ds(srow, STAGE_ROWS), pl.ds(pbase, PHASE_W)],
            spmem.at[pslot, pl.ds(srow, STAGE_ROWS)],
            stsem,
        )

    stage(0, 0).start()
    stage(0, 0).wait()
    plsc.subcore_barrier()

    def phase_body(phase, pcarry):
        pslot = lax.rem(phase, 2)
        pbase = (core * NUM_PHASES + phase) * PHASE_W
        stage(lax.min(phase + 1, NUM_PHASES - 1), 1 - pslot).start()

        def fire_a(g):
            ivec = idx_v[pl.ds(g * LANES, LANES)]
            for lane in range(LANES):
                s = ivec[lane]
                row = base + g * LANES + lane
                pltpu.make_async_copy(
                    spmem.at[pslot, pl.ds(s, 1), pl.ds(0, PW_A)],
                    out_hbm.at[pl.ds(row, 1), pl.ds(pbase, PW_A)],
                    asem,
                ).start()

        def drain_a():
            for _ in range(LANES):
                pltpu.make_async_copy(
                    spmem.at[pslot, pl.ds(0, 1), pl.ds(0, PW_A)],
                    out_hbm.at[pl.ds(base, 1), pl.ds(pbase, PW_A)],
                    asem,
                ).wait()

        def fire_b(g, slot):
            ivec = idx_v[pl.ds(g * LANES, LANES)]
            for lane in range(LANES):
                s = ivec[lane]
                pltpu.make_async_copy(
                    spmem.at[pslot, pl.ds(s, 1), pl.ds(PW_A, PW_B)],
                    bbufs.at[slot, pl.ds(lane, 1)],
                    bsem.at[slot],
                ).start()

        def drain_b(slot):
            for _ in range(LANES):
                pltpu.make_async_copy(
                    spmem.at[pslot, pl.ds(0, 1), pl.ds(PW_A, PW_B)],
                    bbufs.at[0, pl.ds(0, 1)],
                    bsem.at[slot],
                ).wait()

        def b_store(g, slot):
            gc = lax.min(g, NUM_GROUPS - 1)
            return pltpu.make_async_copy(
                bbufs.at[slot],
                out_hbm.at[
                    pl.ds(base + gc * LANES, LANES), pl.ds(pbase + PW_A, PW_B)
                ],
                ssem.at[slot],
            )

        fire_a(0)
        fire_b(0, 0)

        def step(g, carry):
            slot = lax.rem(g, 2)

            @pl.when(g >= 1)
            def _():
                b_store(g - 1, 1 - slot).wait()

            @pl.when(g + 1 < NUM_GROUPS)
            def _():
                fire_b(g + 1, 1 - slot)

            fire_a(lax.min(g + 1, NUM_GROUPS - 1))
            drain_b(slot)
            b_store(g, slot).start()
            drain_a()
            return carry

        lax.fori_loop(0, NUM_GROUPS, step, 0)
        drain_a()
        b_store(NUM_GROUPS - 1, lax.rem(NUM_GROUPS - 1, 2)).wait()
        stage(lax.min(phase + 1, NUM_PHASES - 1), 1 - pslot).wait()
        plsc.subcore_barrier()
        return pcarry

    lax.fori_loop(0, NUM_PHASES, phase_body, 0)


@functools.partial(
    pl.kernel,
    out_type=jax.ShapeDtypeStruct((NUM_ROWS, EMBED_DIM), jnp.float32),
    mesh=plsc.VectorSubcoreMesh(core_axis_name="c", subcore_axis_name="s"),
    scratch_types=[
        pltpu.VMEM((ROWS_PER_WORKER,), jnp.int32),
        pltpu.VMEM((2, LANES, PW_B), jnp.float32),
        pltpu.VMEM_SHARED((2, TBL_ROWS, PHASE_W), jnp.float32),
        pltpu.SemaphoreType.DMA,
        pltpu.SemaphoreType.DMA((2,)),
        pltpu.SemaphoreType.DMA((2,)),
        pltpu.SemaphoreType.DMA,
    ],
)
def _gather_rows(pref_hbm, tbl_hbm, out_hbm, idx_v, bbufs, spmem, asem, bsem, ssem, stsem):
    _sc_body(pref_hbm, tbl_hbm, out_hbm, idx_v, bbufs, spmem, asem, bsem, ssem, stsem)


def kernel(prefix, embedding_table):
    flat_idx = prefix.reshape(NUM_ROWS).astype(jnp.int32)
    out = _gather_rows(flat_idx, embedding_table)
    return out.reshape(BATCH, PRE_SEQ_LEN, EMBED_DIM)


# speedup vs baseline: 1.0064x; 1.0064x over previous
import functools

import jax
import jax.numpy as jnp
from jax import lax
from jax.experimental import pallas as pl
from jax.experimental.pallas import tpu as pltpu
from jax.experimental.pallas import tpu_sc as plsc

PRE_SEQ_LEN = 64
EMBED_DIM = 49152
BATCH = 32
NUM_ROWS = BATCH * PRE_SEQ_LEN
TBL_ROWS = PRE_SEQ_LEN

NUM_CORES = 2
NUM_SUBCORES = 16
LANES = 16

NUM_PHASES = 8
PHASE_W = EMBED_DIM // (NUM_CORES * NUM_PHASES)
PW_A = 1792
PW_B = PHASE_W - PW_A
STAGE_ROWS = TBL_ROWS // NUM_SUBCORES

ROWS_PER_WORKER = NUM_ROWS // NUM_SUBCORES
NUM_GROUPS = ROWS_PER_WORKER // LANES


def _sc_body(pref_hbm, tbl_hbm, out_hbm, idx_v, bbufs, spmem, asem, bsem, ssem, stsem):
    core = lax.axis_index("c")
    sub = lax.axis_index("s")
    base = sub * ROWS_PER_WORKER
    pltpu.sync_copy(pref_hbm.at[pl.ds(base, ROWS_PER_WORKER)], idx_v)
    srow = sub * STAGE_ROWS

    def stage(phase, pslot):
        pbase = (core * NUM_PHASES + phase) * PHASE_W
        return pltpu.make_async_copy(
            tbl_hbm.at[pl.ds(srow, STAGE_ROWS), pl.ds(pbase, PHASE_W)],
            spmem.at[pslot, pl.ds(srow, STAGE_ROWS)],
            stsem,
        )

    stage(0, 0).start()
    stage(0, 0).wait()
    plsc.subcore_barrier()

    def phase_body(phase, pcarry):
        pslot = lax.rem(phase, 2)
        pbase = (core * NUM_PHASES + phase) * PHASE_W
        stage(lax.min(phase + 1, NUM_PHASES - 1), 1 - pslot).start()

        def fire_a(g):
            ivec = idx_v[pl.ds(g * LANES, LANES)]
            for lane in range(LANES):
                s = ivec[lane]
                row = base + g * LANES + lane
                pltpu.make_async_copy(
                    spmem.at[pslot, pl.ds(s, 1), pl.ds(0, PW_A)],
                    out_hbm.at[pl.ds(row, 1), pl.ds(pbase, PW_A)],
                    asem,
                ).start()

        def drain_a():
            for _ in range(LANES):
                pltpu.make_async_copy(
                    spmem.at[pslot, pl.ds(0, 1), pl.ds(0, PW_A)],
                    out_hbm.at[pl.ds(base, 1), pl.ds(pbase, PW_A)],
                    asem,
                ).wait()

        def fire_b(g, slot):
            ivec = idx_v[pl.ds(g * LANES, LANES)]
            for lane in range(LANES):
                s = ivec[lane]
                pltpu.make_async_copy(
                    spmem.at[pslot, pl.ds(s, 1), pl.ds(PW_A, PW_B)],
                    bbufs.at[slot, pl.ds(lane, 1)],
                    bsem.at[slot],
                ).start()

        def drain_b(slot):
            for _ in range(LANES):
                pltpu.make_async_copy(
                    spmem.at[pslot, pl.ds(0, 1), pl.ds(PW_A, PW_B)],
                    bbufs.at[0, pl.ds(0, 1)],
                    bsem.at[slot],
                ).wait()

        def b_store(g, slot):
            gc = lax.min(g, NUM_GROUPS - 1)
            return pltpu.make_async_copy(
                bbufs.at[slot],
                out_hbm.at[
                    pl.ds(base + gc * LANES, LANES), pl.ds(pbase + PW_A, PW_B)
                ],
                ssem.at[slot],
            )

        fire_a(0)
        fire_b(0, 0)

        def step(g, carry):
            slot = lax.rem(g, 2)

            @pl.when(g >= 1)
            def _():
                b_store(g - 1, 1 - slot).wait()

            @pl.when(g + 1 < NUM_GROUPS)
            def _():
                fire_b(g + 1, 1 - slot)

            fire_a(lax.min(g + 1, NUM_GROUPS - 1))
            drain_b(slot)
            b_store(g, slot).start()
            drain_a()
            return carry

        lax.fori_loop(0, NUM_GROUPS, step, 0)
        drain_a()
        b_store(NUM_GROUPS - 1, lax.rem(NUM_GROUPS - 1, 2)).wait()
        stage(lax.min(phase + 1, NUM_PHASES - 1), 1 - pslot).wait()
        plsc.subcore_barrier()
        return pcarry

    lax.fori_loop(0, NUM_PHASES, phase_body, 0)


@functools.partial(
    pl.kernel,
    out_type=jax.ShapeDtypeStruct((NUM_ROWS, EMBED_DIM), jnp.float32),
    mesh=plsc.VectorSubcoreMesh(core_axis_name="c", subcore_axis_name="s"),
    scratch_types=[
        pltpu.VMEM((ROWS_PER_WORKER,), jnp.int32),
        pltpu.VMEM((2, LANES, PW_B), jnp.float32),
        pltpu.VMEM_SHARED((2, TBL_ROWS, PHASE_W), jnp.float32),
        pltpu.SemaphoreType.DMA,
        pltpu.SemaphoreType.DMA((2,)),
        pltpu.SemaphoreType.DMA((2,)),
        pltpu.SemaphoreType.DMA,
    ],
)
def _gather_rows(pref_hbm, tbl_hbm, out_hbm, idx_v, bbufs, spmem, asem, bsem, ssem, stsem):
    _sc_body(pref_hbm, tbl_hbm, out_hbm, idx_v, bbufs, spmem, asem, bsem, ssem, stsem)


def kernel(prefix, embedding_table):
    flat_idx = prefix.reshape(NUM_ROWS).astype(jnp.int32)
    out = _gather_rows(flat_idx, embedding_table)
    return out.reshape(BATCH, PRE_SEQ_LEN, EMBED_DIM)


# speedup vs baseline: 1.0667x; 1.0599x over previous
import functools

import jax
import jax.numpy as jnp
from jax import lax
from jax.experimental import pallas as pl
from jax.experimental.pallas import tpu as pltpu
from jax.experimental.pallas import tpu_sc as plsc

PRE_SEQ_LEN = 64
EMBED_DIM = 49152
BATCH = 32
NUM_ROWS = BATCH * PRE_SEQ_LEN
TBL_ROWS = PRE_SEQ_LEN

NUM_CORES = 2
NUM_SUBCORES = 16
LANES = 16

NUM_PHASES = 8
PHASE_W = EMBED_DIM // (NUM_CORES * NUM_PHASES)
PW_A = 1920
PW_B = PHASE_W - PW_A
STAGE_ROWS = TBL_ROWS // NUM_SUBCORES

ROWS_PER_WORKER = NUM_ROWS // NUM_SUBCORES
NUM_GROUPS = ROWS_PER_WORKER // LANES


def _sc_body(pref_hbm, tbl_hbm, out_hbm, idx_v, bbufs, spmem, asem, bsem, ssem, stsem):
    core = lax.axis_index("c")
    sub = lax.axis_index("s")
    base = sub * ROWS_PER_WORKER
    pltpu.sync_copy(pref_hbm.at[pl.ds(base, ROWS_PER_WORKER)], idx_v)
    srow = sub * STAGE_ROWS

    def stage(phase, pslot):
        pbase = (core * NUM_PHASES + phase) * PHASE_W
        return pltpu.make_async_copy(
            tbl_hbm.at[pl.ds(srow, STAGE_ROWS), pl.ds(pbase, PHASE_W)],
            spmem.at[pslot, pl.ds(srow, STAGE_ROWS)],
            stsem,
        )

    stage(0, 0).start()
    stage(0, 0).wait()
    plsc.subcore_barrier()

    def phase_body(phase, pcarry):
        pslot = lax.rem(phase, 2)
        pbase = (core * NUM_PHASES + phase) * PHASE_W
        stage(lax.min(phase + 1, NUM_PHASES - 1), 1 - pslot).start()

        def fire_a(g):
            ivec = idx_v[pl.ds(g * LANES, LANES)]
            for lane in range(LANES):
                s = ivec[lane]
                row = base + g * LANES + lane
                pltpu.make_async_copy(
                    spmem.at[pslot, pl.ds(s, 1), pl.ds(0, PW_A)],
                    out_hbm.at[pl.ds(row, 1), pl.ds(pbase, PW_A)],
                    asem,
                ).start()

        def drain_a():
            for _ in range(LANES):
                pltpu.make_async_copy(
                    spmem.at[pslot, pl.ds(0, 1), pl.ds(0, PW_A)],
                    out_hbm.at[pl.ds(base, 1), pl.ds(pbase, PW_A)],
                    asem,
                ).wait()

        def fire_b(g, slot):
            ivec = idx_v[pl.ds(g * LANES, LANES)]
            for lane in range(LANES):
                s = ivec[lane]
                pltpu.make_async_copy(
                    spmem.at[pslot, pl.ds(s, 1), pl.ds(PW_A, PW_B)],
                    bbufs.at[slot, pl.ds(lane, 1)],
                    bsem.at[slot],
                ).start()

        def drain_b(slot):
            for _ in range(LANES):
                pltpu.make_async_copy(
                    spmem.at[pslot, pl.ds(0, 1), pl.ds(PW_A, PW_B)],
                    bbufs.at[0, pl.ds(0, 1)],
                    bsem.at[slot],
                ).wait()

        def b_store(g, slot):
            gc = lax.min(g, NUM_GROUPS - 1)
            return pltpu.make_async_copy(
                bbufs.at[slot],
                out_hbm.at[
                    pl.ds(base + gc * LANES, LANES), pl.ds(pbase + PW_A, PW_B)
                ],
                ssem.at[slot],
            )

        fire_a(0)
        fire_b(0, 0)

        def step(g, carry):
            slot = lax.rem(g, 2)

            @pl.when(g >= 1)
            def _():
                b_store(g - 1, 1 - slot).wait()

            @pl.when(g + 1 < NUM_GROUPS)
            def _():
                fire_b(g + 1, 1 - slot)

            fire_a(lax.min(g + 1, NUM_GROUPS - 1))
            drain_b(slot)
            b_store(g, slot).start()
            drain_a()
            return carry

        lax.fori_loop(0, NUM_GROUPS, step, 0)
        drain_a()
        b_store(NUM_GROUPS - 1, lax.rem(NUM_GROUPS - 1, 2)).wait()
        stage(lax.min(phase + 1, NUM_PHASES - 1), 1 - pslot).wait()
        plsc.subcore_barrier()
        return pcarry

    lax.fori_loop(0, NUM_PHASES, phase_body, 0)


@functools.partial(
    pl.kernel,
    out_type=jax.ShapeDtypeStruct((NUM_ROWS, EMBED_DIM), jnp.float32),
    mesh=plsc.VectorSubcoreMesh(core_axis_name="c", subcore_axis_name="s"),
    scratch_types=[
        pltpu.VMEM((ROWS_PER_WORKER,), jnp.int32),
        pltpu.VMEM((2, LANES, PW_B), jnp.float32),
        pltpu.VMEM_SHARED((2, TBL_ROWS, PHASE_W), jnp.float32),
        pltpu.SemaphoreType.DMA,
        pltpu.SemaphoreType.DMA((2,)),
        pltpu.SemaphoreType.DMA((2,)),
        pltpu.SemaphoreType.DMA,
    ],
)
def _gather_rows(pref_hbm, tbl_hbm, out_hbm, idx_v, bbufs, spmem, asem, bsem, ssem, stsem):
    _sc_body(pref_hbm, tbl_hbm, out_hbm, idx_v, bbufs, spmem, asem, bsem, ssem, stsem)


def kernel(prefix, embedding_table):
    flat_idx = prefix.reshape(NUM_ROWS).astype(jnp.int32)
    out = _gather_rows(flat_idx, embedding_table)
    return out.reshape(BATCH, PRE_SEQ_LEN, EMBED_DIM)
